# Initial kernel scaffold; baseline (speedup 1.0000x reference)
import jax
import jax.numpy as jnp
from jax.experimental import pallas as pl


def kernel(h, attention_query, W_att):
    raise NotImplementedError("write your pallas kernel here")


# speedup vs baseline: 2.6586x; 2.6586x over previous
import jax
import jax.numpy as jnp
from jax import lax
from jax.experimental import pallas as pl
from jax.experimental.pallas import tpu as pltpu
from jax.experimental.pallas import tpu_sc as plsc

B = 4096
N = 64
HID = 128
K = 16
BK = B * K

GA = 16


def _topk_idx_body(h_ref, idx_ref):
    h = h_ref[...]
    keys = jnp.max(h, axis=2)
    km = keys[:, None, :]
    kn = keys[:, :, None]
    m_idx = lax.broadcasted_iota(jnp.int32, (GA, N, N), 2)
    n_idx = lax.broadcasted_iota(jnp.int32, (GA, N, N), 1)
    beats = (km > kn) | ((km == kn) & (m_idx < n_idx))
    rank = jnp.sum(beats.astype(jnp.int32), axis=2)
    k_iota = lax.broadcasted_iota(jnp.int32, (GA, K, N), 1)
    onehot = (rank[:, None, :] == k_iota).astype(jnp.int32)
    node = lax.broadcasted_iota(jnp.int32, (GA, K, N), 2)
    nsel = jnp.sum(onehot * node, axis=2)
    g0 = pl.program_id(0) * GA
    graph = g0 + lax.broadcasted_iota(jnp.int32, (GA, K), 0)
    idx_ref[...] = graph * N + nsel


def _topk_idx(h):
    return pl.pallas_call(
        _topk_idx_body,
        grid=(B // GA,),
        in_specs=[pl.BlockSpec((GA, N, HID), lambda i: (i, 0, 0))],
        out_specs=pl.BlockSpec((GA, K), lambda i: (i, 0)),
        out_shape=jax.ShapeDtypeStruct((B, K), jnp.int32),
    )(h)


NC, NS = 2, 16
NW = NC * NS
ROWS_PER_W = BK // NW
CH = 128
NCH = ROWS_PER_W // CH


def _sc_gather_body(tbl_ref, idx_ref, out_ref, idx_v, rows_v, sem):
    wid = lax.axis_index("s") * NC + lax.axis_index("c")
    base = wid * ROWS_PER_W
    for i in range(NCH):
        off = base + i * CH
        pltpu.sync_copy(idx_ref.at[pl.ds(off, CH)], idx_v)
        pltpu.async_copy(tbl_ref.at[idx_v], rows_v, sem).wait()
        pltpu.sync_copy(rows_v, out_ref.at[pl.ds(off, CH)])


import functools


@functools.lru_cache(maxsize=1)
def _sc_gather():
    return pl.kernel(
        _sc_gather_body,
        out_type=jax.ShapeDtypeStruct((BK, HID), jnp.float32),
        mesh=plsc.VectorSubcoreMesh(core_axis_name="c", subcore_axis_name="s"),
        scratch_types=[
            pltpu.VMEM((CH,), jnp.int32),
            pltpu.VMEM((CH, HID), jnp.float32),
            pltpu.SemaphoreType.DMA,
        ],
    )


GC = 32
R = GC * K


def _lane_bitonic_sort(x):
    lane = lax.broadcasted_iota(jnp.int32, (1, HID), 1)
    size = 2
    while size <= HID:
        stride = size // 2
        while stride >= 1:
            upper = (lane & stride) != 0
            take_min = jnp.logical_xor((lane & stride) == 0,
                                       (lane & size) != 0)
            p = jnp.where(upper, pltpu.roll(x, stride, 1),
                          pltpu.roll(x, HID - stride, 1))
            x = jnp.where(take_min, jnp.minimum(x, p), jnp.maximum(x, p))
            stride //= 2
        size *= 2
    return x


def _sort_attn_body(x_ref, q_ref, w1_ref, w2_ref, out_ref):
    x = _lane_bitonic_sort(x_ref[...])
    x3 = x.reshape(GC, K, HID)
    w1 = w1_ref[...].reshape(1, 1, HID)
    dot = jnp.sum(x3 * w1, axis=2)
    q = q_ref[...]
    qdot = jnp.sum(q * w2_ref[...], axis=1, keepdims=True)
    logit = dot + qdot
    logit = jnp.where(logit >= 0, logit, 0.01 * logit)
    mx = jnp.max(logit, axis=1, keepdims=True)
    e = jnp.exp(logit - mx)
    wgt = e / jnp.sum(e, axis=1, keepdims=True)
    out_ref[...] = jnp.sum(x3 * wgt[:, :, None], axis=1)


def _sort_attn(pooled, attention_query, w1, w2):
    return pl.pallas_call(
        _sort_attn_body,
        grid=(B // GC,),
        in_specs=[
            pl.BlockSpec((R, HID), lambda i: (i, 0)),
            pl.BlockSpec((GC, HID), lambda i: (i, 0)),
            pl.BlockSpec((1, HID), lambda i: (0, 0)),
            pl.BlockSpec((1, HID), lambda i: (0, 0)),
        ],
        out_specs=pl.BlockSpec((GC, HID), lambda i: (i, 0)),
        out_shape=jax.ShapeDtypeStruct((B, HID), jnp.float32),
    )(pooled, attention_query, w1, w2)


def kernel(h, attention_query, W_att):
    idx = _topk_idx(h)
    pooled = _sc_gather()(h.reshape(B * N, HID), idx.reshape(BK))
    w1 = W_att[:HID, 0].reshape(1, HID)
    w2 = W_att[HID:, 0].reshape(1, HID)
    return _sort_attn(pooled, attention_query, w1, w2)


# speedup vs baseline: 3.1323x; 1.1782x over previous
import jax
import jax.numpy as jnp
from jax import lax
from jax.experimental import pallas as pl
from jax.experimental.pallas import tpu as pltpu
from jax.experimental.pallas import tpu_sc as plsc

B = 4096
N = 64
HID = 128
K = 16
BK = B * K

GA = 16


def _topk_idx_body(h_ref, idx_ref):
    h = h_ref[...]
    keys = jnp.max(h, axis=2)
    neg = jnp.full((GA, N), -jnp.inf, dtype=keys.dtype)
    kv = jnp.concatenate([keys, neg], axis=1)
    nid = lax.broadcasted_iota(jnp.int32, (GA, 2 * N), 1)
    lane = lax.broadcasted_iota(jnp.int32, (1, HID), 1)
    size = 2
    while size <= HID:
        stride = size // 2
        while stride >= 1:
            upper = (lane & stride) != 0
            keep_small = jnp.logical_xor((lane & stride) == 0,
                                         (lane & size) != 0)
            kp = jnp.where(upper, pltpu.roll(kv, stride, 1),
                           pltpu.roll(kv, HID - stride, 1))
            np_ = jnp.where(upper, pltpu.roll(nid, stride, 1),
                            pltpu.roll(nid, HID - stride, 1))
            self_small = (kv < kp) | ((kv == kp) & (nid > np_))
            keep_self = self_small == keep_small
            kv = jnp.where(keep_self, kv, kp)
            nid = jnp.where(keep_self, nid, np_)
            stride //= 2
        size *= 2
    g0 = pl.program_id(0) * GA
    graph = g0 + lax.broadcasted_iota(jnp.int32, (GA, K), 0)
    idx_ref[...] = graph * N + nid[:, HID - K:]


def _topk_idx(h):
    return pl.pallas_call(
        _topk_idx_body,
        grid=(B // GA,),
        in_specs=[pl.BlockSpec((GA, N, HID), lambda i: (i, 0, 0))],
        out_specs=pl.BlockSpec((GA, K), lambda i: (i, 0)),
        out_shape=jax.ShapeDtypeStruct((B, K), jnp.int32),
    )(h)


NC, NS = 2, 16
NW = NC * NS
ROWS_PER_W = BK // NW
CH = 128
NCH = ROWS_PER_W // CH


def _sc_gather_body(tbl_ref, idx_ref, out_ref, idx_v, rows_v, sem):
    wid = lax.axis_index("s") * NC + lax.axis_index("c")
    base = wid * ROWS_PER_W
    for i in range(NCH):
        off = base + i * CH
        pltpu.sync_copy(idx_ref.at[pl.ds(off, CH)], idx_v)
        pltpu.async_copy(tbl_ref.at[idx_v], rows_v, sem).wait()
        pltpu.sync_copy(rows_v, out_ref.at[pl.ds(off, CH)])


import functools


@functools.lru_cache(maxsize=1)
def _sc_gather():
    return pl.kernel(
        _sc_gather_body,
        out_type=jax.ShapeDtypeStruct((BK, HID), jnp.float32),
        mesh=plsc.VectorSubcoreMesh(core_axis_name="c", subcore_axis_name="s"),
        scratch_types=[
            pltpu.VMEM((CH,), jnp.int32),
            pltpu.VMEM((CH, HID), jnp.float32),
            pltpu.SemaphoreType.DMA,
        ],
    )


GC = 32
R = GC * K


def _lane_bitonic_sort(x):
    lane = lax.broadcasted_iota(jnp.int32, (1, HID), 1)
    size = 2
    while size <= HID:
        stride = size // 2
        while stride >= 1:
            upper = (lane & stride) != 0
            take_min = jnp.logical_xor((lane & stride) == 0,
                                       (lane & size) != 0)
            p = jnp.where(upper, pltpu.roll(x, stride, 1),
                          pltpu.roll(x, HID - stride, 1))
            x = jnp.where(take_min, jnp.minimum(x, p), jnp.maximum(x, p))
            stride //= 2
        size *= 2
    return x


GCH = 4
RCH = GCH * K


def _sort_attn_body(x_ref, q_ref, w1_ref, w2_ref, out_ref):
    w1 = w1_ref[...].reshape(1, 1, HID)
    w2 = w2_ref[...]

    def chunk(c, _):
        x = _lane_bitonic_sort(x_ref[pl.ds(c * RCH, RCH), :])
        x3 = x.reshape(GCH, K, HID)
        dot = jnp.sum(x3 * w1, axis=2)
        q = q_ref[pl.ds(c * GCH, GCH), :]
        qdot = jnp.sum(q * w2, axis=1, keepdims=True)
        logit = dot + qdot
        logit = jnp.where(logit >= 0, logit, 0.01 * logit)
        mx = jnp.max(logit, axis=1, keepdims=True)
        e = jnp.exp(logit - mx)
        wgt = e / jnp.sum(e, axis=1, keepdims=True)
        out_ref[pl.ds(c * GCH, GCH), :] = jnp.sum(x3 * wgt[:, :, None], axis=1)
        return 0

    lax.fori_loop(0, GC // GCH, chunk, 0)


def _sort_attn(pooled, attention_query, w1, w2):
    return pl.pallas_call(
        _sort_attn_body,
        grid=(B // GC,),
        in_specs=[
            pl.BlockSpec((R, HID), lambda i: (i, 0)),
            pl.BlockSpec((GC, HID), lambda i: (i, 0)),
            pl.BlockSpec((1, HID), lambda i: (0, 0)),
            pl.BlockSpec((1, HID), lambda i: (0, 0)),
        ],
        out_specs=pl.BlockSpec((GC, HID), lambda i: (i, 0)),
        out_shape=jax.ShapeDtypeStruct((B, HID), jnp.float32),
    )(pooled, attention_query, w1, w2)


def kernel(h, attention_query, W_att):
    idx = _topk_idx(h)
    pooled = _sc_gather()(h.reshape(B * N, HID), idx.reshape(BK))
    w1 = W_att[:HID, 0].reshape(1, HID)
    w2 = W_att[HID:, 0].reshape(1, HID)
    return _sort_attn(pooled, attention_query, w1, w2)


# speedup vs baseline: 6.3754x; 2.0354x over previous
import jax
import jax.numpy as jnp
from jax import lax
from jax.experimental import pallas as pl
from jax.experimental.pallas import tpu as pltpu
from jax.experimental.pallas import tpu_sc as plsc

B = 4096
N = 64
HID = 128
K = 16
BK = B * K

GA = 64


def _topk_idx_body(h_ref, idx_ref):
    h = h_ref[...]
    keys = jnp.max(h, axis=2)
    neg = jnp.full((GA, N), -jnp.inf, dtype=keys.dtype)
    kv = jnp.concatenate([keys, neg], axis=1)
    nid = lax.broadcasted_iota(jnp.int32, (GA, 2 * N), 1)
    lane = lax.broadcasted_iota(jnp.int32, (1, HID), 1)
    size = 2
    while size <= HID:
        stride = size // 2
        while stride >= 1:
            upper = (lane & stride) != 0
            keep_small = jnp.logical_xor((lane & stride) == 0,
                                         (lane & size) != 0)
            kp = jnp.where(upper, pltpu.roll(kv, stride, 1),
                           pltpu.roll(kv, HID - stride, 1))
            np_ = jnp.where(upper, pltpu.roll(nid, stride, 1),
                            pltpu.roll(nid, HID - stride, 1))
            self_small = (kv < kp) | ((kv == kp) & (nid > np_))
            keep_self = self_small == keep_small
            kv = jnp.where(keep_self, kv, kp)
            nid = jnp.where(keep_self, nid, np_)
            stride //= 2
        size *= 2
    g0 = pl.program_id(0) * GA
    graph = g0 + lax.broadcasted_iota(jnp.int32, (GA, K), 0)
    idx_ref[...] = graph * N + nid[:, HID - K:]


def _topk_idx(h):
    return pl.pallas_call(
        _topk_idx_body,
        grid=(B // GA,),
        in_specs=[pl.BlockSpec((GA, N, HID), lambda i: (i, 0, 0))],
        out_specs=pl.BlockSpec((GA, K), lambda i: (i, 0)),
        out_shape=jax.ShapeDtypeStruct((B, K), jnp.int32),
    )(h)


NC, NS = 2, 16
NW = NC * NS
ROWS_PER_W = BK // NW
CH = 128
NCH = ROWS_PER_W // CH


def _sc_gather_body(tbl_ref, idx_ref, out_ref, idx_v, rows_v, sem):
    wid = lax.axis_index("s") * NC + lax.axis_index("c")
    base = wid * ROWS_PER_W
    for i in range(NCH):
        off = base + i * CH
        pltpu.sync_copy(idx_ref.at[pl.ds(off, CH)], idx_v)
        pltpu.async_copy(tbl_ref.at[idx_v], rows_v, sem).wait()
        pltpu.sync_copy(rows_v, out_ref.at[pl.ds(off, CH)])


import functools


@functools.lru_cache(maxsize=1)
def _sc_gather():
    return pl.kernel(
        _sc_gather_body,
        out_type=jax.ShapeDtypeStruct((BK, HID), jnp.float32),
        mesh=plsc.VectorSubcoreMesh(core_axis_name="c", subcore_axis_name="s"),
        scratch_types=[
            pltpu.VMEM((CH,), jnp.int32),
            pltpu.VMEM((CH, HID), jnp.float32),
            pltpu.SemaphoreType.DMA,
        ],
    )


GC = 32
R = GC * K


def _lane_bitonic_sort(x):
    lane = lax.broadcasted_iota(jnp.int32, (1, HID), 1)
    size = 2
    while size <= HID:
        stride = size // 2
        while stride >= 1:
            upper = (lane & stride) != 0
            take_min = jnp.logical_xor((lane & stride) == 0,
                                       (lane & size) != 0)
            p = jnp.where(upper, pltpu.roll(x, stride, 1),
                          pltpu.roll(x, HID - stride, 1))
            x = jnp.where(take_min, jnp.minimum(x, p), jnp.maximum(x, p))
            stride //= 2
        size *= 2
    return x


GCH = 8
RCH = GCH * K


def _sort_attn_body(x_ref, q_ref, w1_ref, w2_ref, out_ref):
    w1 = w1_ref[...].reshape(1, 1, HID)
    w2 = w2_ref[...]

    def chunk(c, _):
        x = _lane_bitonic_sort(x_ref[pl.ds(c * RCH, RCH), :])
        x3 = x.reshape(GCH, K, HID)
        dot = jnp.sum(x3 * w1, axis=2)
        q = q_ref[pl.ds(c * GCH, GCH), :]
        qdot = jnp.sum(q * w2, axis=1, keepdims=True)
        logit = dot + qdot
        logit = jnp.where(logit >= 0, logit, 0.01 * logit)
        mx = jnp.max(logit, axis=1, keepdims=True)
        e = jnp.exp(logit - mx)
        wgt = e / jnp.sum(e, axis=1, keepdims=True)
        out_ref[pl.ds(c * GCH, GCH), :] = jnp.sum(x3 * wgt[:, :, None], axis=1)
        return 0

    lax.fori_loop(0, GC // GCH, chunk, 0)


def _sort_attn(pooled, attention_query, w1, w2):
    return pl.pallas_call(
        _sort_attn_body,
        grid=(B // GC,),
        in_specs=[
            pl.BlockSpec((R, HID), lambda i: (i, 0)),
            pl.BlockSpec((GC, HID), lambda i: (i, 0)),
            pl.BlockSpec((1, HID), lambda i: (0, 0)),
            pl.BlockSpec((1, HID), lambda i: (0, 0)),
        ],
        out_specs=pl.BlockSpec((GC, HID), lambda i: (i, 0)),
        out_shape=jax.ShapeDtypeStruct((B, HID), jnp.float32),
    )(pooled, attention_query, w1, w2)


def kernel(h, attention_query, W_att):
    idx = _topk_idx(h)
    pooled = _sc_gather()(h.reshape(B * N, HID), idx.reshape(BK))
    w1 = W_att[:HID, 0].reshape(1, HID)
    w2 = W_att[HID:, 0].reshape(1, HID)
    return _sort_attn(pooled, attention_query, w1, w2)


# speedup vs baseline: 9.5340x; 1.4954x over previous
import jax
import jax.numpy as jnp
from jax import lax
from jax.experimental import pallas as pl
from jax.experimental.pallas import tpu as pltpu
from jax.experimental.pallas import tpu_sc as plsc

B = 4096
N = 64
HID = 128
K = 16
BK = B * K

GA = 128


def _topk_idx_body(h_ref, idx_ref):
    h = h_ref[...]
    keys = jnp.max(h, axis=2)
    neg = jnp.full((GA, N), -jnp.inf, dtype=keys.dtype)
    kv = jnp.concatenate([keys, neg], axis=1)
    nid = lax.broadcasted_iota(jnp.int32, (GA, 2 * N), 1)
    lane = lax.broadcasted_iota(jnp.int32, (1, HID), 1)
    size = 2
    while size <= HID:
        stride = size // 2
        while stride >= 1:
            upper = (lane & stride) != 0
            keep_small = jnp.logical_xor((lane & stride) == 0,
                                         (lane & size) != 0)
            kp = jnp.where(upper, pltpu.roll(kv, stride, 1),
                           pltpu.roll(kv, HID - stride, 1))
            np_ = jnp.where(upper, pltpu.roll(nid, stride, 1),
                            pltpu.roll(nid, HID - stride, 1))
            self_small = (kv < kp) | ((kv == kp) & (nid > np_))
            keep_self = self_small == keep_small
            kv = jnp.where(keep_self, kv, kp)
            nid = jnp.where(keep_self, nid, np_)
            stride //= 2
        size *= 2
    g0 = pl.program_id(0) * GA
    graph = g0 + lax.broadcasted_iota(jnp.int32, (GA, K), 0)
    idx_ref[...] = graph * N + nid[:, HID - K:]


def _topk_idx(h):
    return pl.pallas_call(
        _topk_idx_body,
        grid=(B // GA,),
        in_specs=[pl.BlockSpec((GA, N, HID), lambda i: (i, 0, 0))],
        out_specs=pl.BlockSpec((GA, K), lambda i: (i, 0)),
        out_shape=jax.ShapeDtypeStruct((B, K), jnp.int32),
    )(h)


NC, NS = 2, 16
NW = NC * NS
ROWS_PER_W = BK // NW
CH = 128
NCH = ROWS_PER_W // CH


def _sc_gather_body(tbl_ref, idx_ref, out_ref, idx_v, rows_v, sem):
    wid = lax.axis_index("s") * NC + lax.axis_index("c")
    base = wid * ROWS_PER_W
    for i in range(NCH):
        off = base + i * CH
        pltpu.sync_copy(idx_ref.at[pl.ds(off, CH)], idx_v)
        pltpu.async_copy(tbl_ref.at[idx_v], rows_v, sem).wait()
        pltpu.sync_copy(rows_v, out_ref.at[pl.ds(off, CH)])


import functools


@functools.lru_cache(maxsize=1)
def _sc_gather():
    return pl.kernel(
        _sc_gather_body,
        out_type=jax.ShapeDtypeStruct((BK, HID), jnp.float32),
        mesh=plsc.VectorSubcoreMesh(core_axis_name="c", subcore_axis_name="s"),
        scratch_types=[
            pltpu.VMEM((CH,), jnp.int32),
            pltpu.VMEM((CH, HID), jnp.float32),
            pltpu.SemaphoreType.DMA,
        ],
    )


GC = 32
R = GC * K


def _lane_bitonic_sort(x):
    lane = lax.broadcasted_iota(jnp.int32, (1, HID), 1)
    size = 2
    while size <= HID:
        stride = size // 2
        while stride >= 1:
            upper = (lane & stride) != 0
            take_min = jnp.logical_xor((lane & stride) == 0,
                                       (lane & size) != 0)
            p = jnp.where(upper, pltpu.roll(x, stride, 1),
                          pltpu.roll(x, HID - stride, 1))
            x = jnp.where(take_min, jnp.minimum(x, p), jnp.maximum(x, p))
            stride //= 2
        size *= 2
    return x


GCH = 8
RCH = GCH * K


def _sort_attn_body(x_ref, q_ref, w1_ref, w2_ref, out_ref):
    w1 = w1_ref[...].reshape(1, 1, HID)
    w2 = w2_ref[...]

    def chunk(c, _):
        x = _lane_bitonic_sort(x_ref[pl.ds(c * RCH, RCH), :])
        x3 = x.reshape(GCH, K, HID)
        dot = jnp.sum(x3 * w1, axis=2)
        q = q_ref[pl.ds(c * GCH, GCH), :]
        qdot = jnp.sum(q * w2, axis=1, keepdims=True)
        logit = dot + qdot
        logit = jnp.where(logit >= 0, logit, 0.01 * logit)
        mx = jnp.max(logit, axis=1, keepdims=True)
        e = jnp.exp(logit - mx)
        wgt = e / jnp.sum(e, axis=1, keepdims=True)
        out_ref[pl.ds(c * GCH, GCH), :] = jnp.sum(x3 * wgt[:, :, None], axis=1)
        return 0

    lax.fori_loop(0, GC // GCH, chunk, 0, unroll=2)


def _sort_attn(pooled, attention_query, w1, w2):
    return pl.pallas_call(
        _sort_attn_body,
        grid=(B // GC,),
        in_specs=[
            pl.BlockSpec((R, HID), lambda i: (i, 0)),
            pl.BlockSpec((GC, HID), lambda i: (i, 0)),
            pl.BlockSpec((1, HID), lambda i: (0, 0)),
            pl.BlockSpec((1, HID), lambda i: (0, 0)),
        ],
        out_specs=pl.BlockSpec((GC, HID), lambda i: (i, 0)),
        out_shape=jax.ShapeDtypeStruct((B, HID), jnp.float32),
    )(pooled, attention_query, w1, w2)


def kernel(h, attention_query, W_att):
    idx = _topk_idx(h)
    pooled = _sc_gather()(h.reshape(B * N, HID), idx.reshape(BK))
    w1 = W_att[:HID, 0].reshape(1, HID)
    w2 = W_att[HID:, 0].reshape(1, HID)
    return _sort_attn(pooled, attention_query, w1, w2)


# speedup vs baseline: 10.0187x; 1.0508x over previous
import jax
import jax.numpy as jnp
from jax import lax
from jax.experimental import pallas as pl
from jax.experimental.pallas import tpu as pltpu
from jax.experimental.pallas import tpu_sc as plsc

B = 4096
N = 64
HID = 128
K = 16
BK = B * K

GA = 128


GA2 = GA // 2


def _topk_idx_body(h_ref, idx_ref):
    h = h_ref[...]
    keys = jnp.max(h, axis=2)
    kv = jnp.concatenate([keys[:GA2], keys[GA2:]], axis=1)
    lane = lax.broadcasted_iota(jnp.int32, (1, HID), 1)
    llane = lane & (N - 1)
    nid = jnp.broadcast_to(llane, (GA2, HID))
    size = 2
    while size <= N:
        stride = size // 2
        while stride >= 1:
            upper = (llane & stride) != 0
            keep_small = jnp.logical_xor((llane & stride) == 0,
                                         (llane & size) != 0)
            kp = jnp.where(upper, pltpu.roll(kv, stride, 1),
                           pltpu.roll(kv, HID - stride, 1))
            np_ = jnp.where(upper, pltpu.roll(nid, stride, 1),
                            pltpu.roll(nid, HID - stride, 1))
            self_small = (kv < kp) | ((kv == kp) & (nid > np_))
            keep_self = self_small == keep_small
            kv = jnp.where(keep_self, kv, kp)
            nid = jnp.where(keep_self, nid, np_)
            stride //= 2
        size *= 2
    g0 = pl.program_id(0) * GA
    graph = g0 + lax.broadcasted_iota(jnp.int32, (GA2, K), 0)
    idx_ref[:GA2, :] = (graph * N) + nid[:, N - K:N]
    idx_ref[GA2:, :] = ((g0 + GA2) * N) + (
        lax.broadcasted_iota(jnp.int32, (GA2, K), 0) * N + nid[:, HID - K:])


def _topk_idx(h):
    return pl.pallas_call(
        _topk_idx_body,
        grid=(B // GA,),
        in_specs=[pl.BlockSpec((GA, N, HID), lambda i: (i, 0, 0))],
        out_specs=pl.BlockSpec((GA, K), lambda i: (i, 0)),
        out_shape=jax.ShapeDtypeStruct((B, K), jnp.int32),
    )(h)


NC, NS = 2, 16
NW = NC * NS
ROWS_PER_W = BK // NW
CH = 128
NCH = ROWS_PER_W // CH


def _sc_gather_body(tbl_ref, idx_ref, out_ref, idx_v, rows_v, sem):
    wid = lax.axis_index("s") * NC + lax.axis_index("c")
    base = wid * ROWS_PER_W
    for i in range(NCH):
        off = base + i * CH
        pltpu.sync_copy(idx_ref.at[pl.ds(off, CH)], idx_v)
        pltpu.async_copy(tbl_ref.at[idx_v], rows_v, sem).wait()
        pltpu.sync_copy(rows_v, out_ref.at[pl.ds(off, CH)])


import functools


@functools.lru_cache(maxsize=1)
def _sc_gather():
    return pl.kernel(
        _sc_gather_body,
        out_type=jax.ShapeDtypeStruct((BK, HID), jnp.float32),
        mesh=plsc.VectorSubcoreMesh(core_axis_name="c", subcore_axis_name="s"),
        scratch_types=[
            pltpu.VMEM((CH,), jnp.int32),
            pltpu.VMEM((CH, HID), jnp.float32),
            pltpu.SemaphoreType.DMA,
        ],
    )


GC = 32
R = GC * K


def _lane_bitonic_sort(x):
    lane = lax.broadcasted_iota(jnp.int32, (1, HID), 1)
    size = 2
    while size <= HID:
        stride = size // 2
        while stride >= 1:
            upper = (lane & stride) != 0
            take_min = jnp.logical_xor((lane & stride) == 0,
                                       (lane & size) != 0)
            p = jnp.where(upper, pltpu.roll(x, stride, 1),
                          pltpu.roll(x, HID - stride, 1))
            x = jnp.where(take_min, jnp.minimum(x, p), jnp.maximum(x, p))
            stride //= 2
        size *= 2
    return x


GCH = 8
RCH = GCH * K


def _sort_attn_body(x_ref, q_ref, w1_ref, w2_ref, out_ref):
    w1 = w1_ref[...].reshape(1, 1, HID)
    w2 = w2_ref[...]

    def chunk(c, _):
        x = _lane_bitonic_sort(x_ref[pl.ds(c * RCH, RCH), :])
        x3 = x.reshape(GCH, K, HID)
        dot = jnp.sum(x3 * w1, axis=2)
        q = q_ref[pl.ds(c * GCH, GCH), :]
        qdot = jnp.sum(q * w2, axis=1, keepdims=True)
        logit = dot + qdot
        logit = jnp.where(logit >= 0, logit, 0.01 * logit)
        mx = jnp.max(logit, axis=1, keepdims=True)
        e = jnp.exp(logit - mx)
        wgt = e / jnp.sum(e, axis=1, keepdims=True)
        out_ref[pl.ds(c * GCH, GCH), :] = jnp.sum(x3 * wgt[:, :, None], axis=1)
        return 0

    lax.fori_loop(0, GC // GCH, chunk, 0, unroll=2)


def _sort_attn(pooled, attention_query, w1, w2):
    return pl.pallas_call(
        _sort_attn_body,
        grid=(B // GC,),
        in_specs=[
            pl.BlockSpec((R, HID), lambda i: (i, 0)),
            pl.BlockSpec((GC, HID), lambda i: (i, 0)),
            pl.BlockSpec((1, HID), lambda i: (0, 0)),
            pl.BlockSpec((1, HID), lambda i: (0, 0)),
        ],
        out_specs=pl.BlockSpec((GC, HID), lambda i: (i, 0)),
        out_shape=jax.ShapeDtypeStruct((B, HID), jnp.float32),
    )(pooled, attention_query, w1, w2)


def kernel(h, attention_query, W_att):
    idx = _topk_idx(h)
    pooled = _sc_gather()(h.reshape(B * N, HID), idx.reshape(BK))
    w1 = W_att[:HID, 0].reshape(1, HID)
    w2 = W_att[HID:, 0].reshape(1, HID)
    return _sort_attn(pooled, attention_query, w1, w2)
